# baseline (speedup 1.0000x reference)
import functools

import jax
import jax.numpy as jnp
from jax import lax
from jax.experimental import pallas as pl
from jax.experimental.pallas import tpu as pltpu
from jax.experimental.pallas import tpu_sc as plsc

_C = 192
_N = 1024
_B = 32
_NC = 2
_NS = 16
_NW = _NC * _NS
_ROWS_PC = 8
_CHUNK = _ROWS_PC * _N
_NCH = _C // _ROWS_PC
_LUT = 256
_MAGIC = 12582912.0

_SP0, _SP1, _SP2, _SP3, _SP4 = 0, 3, 12, 21, 30
_B0, _B1, _B2, _B3, _B4 = 33, 36, 39, 42, 45
_T0, _T1, _T2, _T3 = 46, 49, 52, 55


def _builder_body(wraw_ref, lut_ref, wproc_ref):
    wr = wraw_ref[...]
    col = lax.broadcasted_iota(jnp.int32, wr.shape, 1)
    wp = jnp.where(col < _B0, jax.nn.softplus(wr),
                   jnp.where(col < _T0, wr, jnp.tanh(wr)))
    wproc_ref[...] = wp

    def sl(j):
        return wp[:, j:j + 1]

    def logits(u):
        l = [sl(_SP0 + j) * u + sl(_B0 + j) for j in range(3)]
        l = [l[j] + sl(_T0 + j) * jnp.tanh(l[j]) for j in range(3)]
        for (wc, bc, tc) in ((_SP1, _B1, _T1), (_SP2, _B2, _T2),
                             (_SP3, _B3, _T3)):
            nl = [sl(wc + 3 * j) * l[0] + sl(wc + 3 * j + 1) * l[1]
                  + sl(wc + 3 * j + 2) * l[2] + sl(bc + j) for j in range(3)]
            l = [nl[j] + sl(tc + j) * jnp.tanh(nl[j]) for j in range(3)]
        return sl(_SP4) * l[0] + sl(_SP4 + 1) * l[1] + sl(_SP4 + 2) * l[2] \
            + sl(_B4)

    vh = lax.broadcasted_iota(
        jnp.int32, (wr.shape[0], _LUT), 1).astype(jnp.float32) - 128.0
    lower = logits(vh - 0.5)
    upper = logits(vh + 0.5)
    s = -jnp.sign(lower + upper)
    lk = jnp.abs(jax.nn.sigmoid(s * upper) - jax.nn.sigmoid(s * lower))
    lut_ref[...] = jnp.maximum(lk, 1e-9)


def _build_tables(wraw):
    return pl.pallas_call(
        _builder_body,
        out_shape=[jax.ShapeDtypeStruct((_C, _LUT), jnp.float32),
                   jax.ShapeDtypeStruct((_C, 64), jnp.float32)],
    )(wraw)


def _tanh_e(t):
    return 1.0 - 2.0 / (jnp.exp(2.0 * t) + 1.0)


def _sigmoid_e(t):
    return 1.0 / (1.0 + jnp.exp(-t))


def _exact_round(v):
    m1 = (v + _MAGIC) - _MAGIC
    ki = (2.0 * v).astype(jnp.int32)
    odd = jnp.bitwise_and(ki, 1)
    mid = jnp.where(odd == 1, ((ki + 2) >> 2) << 1, ki >> 1)
    mid = mid.astype(jnp.float32)
    av = jnp.abs(v)
    return jnp.where(av <= 4194304.0, m1,
                     jnp.where(av < 8388608.0, mid, v))


def _mlp(wv, u):
    def g(c):
        return wv[c // 16][c % 16]

    l = [g(_SP0 + j) * u + g(_B0 + j) for j in range(3)]
    l = [l[j] + g(_T0 + j) * _tanh_e(l[j]) for j in range(3)]
    for (wc, bc, tc) in ((_SP1, _B1, _T1), (_SP2, _B2, _T2), (_SP3, _B3, _T3)):
        nl = [g(wc + 3 * j) * l[0] + g(wc + 3 * j + 1) * l[1]
              + g(wc + 3 * j + 2) * l[2] + g(bc + j)
              for j in range(3)]
        l = [nl[j] + g(tc + j) * _tanh_e(nl[j]) for j in range(3)]
    return (g(_SP4) * l[0] + g(_SP4 + 1) * l[1]
            + g(_SP4 + 2) * l[2] + g(_B4))


def _round_body(x_ref, xh_ref):
    xh_ref[...] = jnp.round(x_ref[...])


def _round_tc(x):
    B, C, H, W = x.shape
    cb = 8
    return pl.pallas_call(
        _round_body,
        grid=(B, C // cb),
        in_specs=[pl.BlockSpec((1, cb, H, W), lambda b, c: (b, c, 0, 0))],
        out_specs=pl.BlockSpec((1, cb, H, W), lambda b, c: (b, c, 0, 0)),
        out_shape=jax.ShapeDtypeStruct((B, C, H, W), jnp.float32),
    )(x)


def _sc_body(x_hbm, lut_hbm, w_hbm, lk_hbm,
             lut_v, w_v, xin0, xin1, olk0, olk1,
             si0, si1, sl0, sl1):
    cid = lax.axis_index("c")
    sid = lax.axis_index("s")
    wid = sid * _NC + cid
    base = wid * (_C * _N)
    pltpu.sync_copy(lut_hbm, lut_v)
    pltpu.sync_copy(w_hbm, w_v)

    xin = (xin0, xin1)
    olk = (olk0, olk1)
    sin = (si0, si1)
    slk = (sl0, sl1)

    def start_in(t, b):
        pltpu.async_copy(x_hbm.at[pl.ds(base + t * _CHUNK, _CHUNK)],
                         xin[b], sin[b])

    def compute(t, b):
        xv, lv = xin[b], olk[b]
        r0 = t * _ROWS_PC

        @plsc.parallel_loop(0, _CHUNK // 16, unroll=8,
                            carry=jnp.zeros((16,), jnp.float32))
        def macc(i, acc):
            off = i * 16
            ch = r0 + (i >> 6)
            chb = jnp.broadcast_to(ch << 8, (16,)) + 128
            v = xv[pl.ds(off, 16)]
            acc = jnp.maximum(acc, jnp.abs(v))
            vh = (v + _MAGIC) - _MAGIC
            ci = jnp.clip(vh, -128.0, 127.0).astype(jnp.int32) + chb
            lv[pl.ds(off, 16)] = plsc.load_gather(lut_v, [ci])
            return acc

        m = jnp.max(macc)

        @pl.when(m > 127.0)
        def _slow():
            def row_slow(r, carry2):
                ch = r0 + r
                wv = tuple(w_v[pl.ds(ch * 64 + k * 16, 16)]
                           for k in range(4))
                rb = r * _N

                def slow(i, carry3):
                    off = rb + i * 16
                    v = xv[pl.ds(off, 16)]
                    vh = _exact_round(v)
                    lo = _mlp(wv, vh - 0.5)
                    up = _mlp(wv, vh + 0.5)
                    s = -jnp.sign(lo + up)
                    lk = jnp.abs(_sigmoid_e(s * up) - _sigmoid_e(s * lo))
                    lv[pl.ds(off, 16)] = jnp.maximum(lk, 1e-9)
                    return carry3

                return lax.fori_loop(0, _N // 16, slow, carry2)

            lax.fori_loop(0, _ROWS_PC, row_slow, 0)

    start_in(0, 0)

    def gbody(g, carry):
        for u in (0, 1):
            t = 2 * g + u

            @pl.when(t + 1 < _NCH)
            def _pref():
                start_in(t + 1, 1 - u)

            pltpu.make_async_copy(
                x_hbm.at[pl.ds(0, _CHUNK)], xin[u], sin[u]).wait()

            @pl.when(g >= 1)
            def _wait_out():
                pltpu.make_async_copy(
                    olk[u], lk_hbm.at[pl.ds(0, _CHUNK)], slk[u]).wait()

            compute(t, u)
            pltpu.async_copy(
                olk[u], lk_hbm.at[pl.ds(base + t * _CHUNK, _CHUNK)], slk[u])
        return carry

    lax.fori_loop(0, _NCH // 2, gbody, 0)
    for u in (0, 1):
        pltpu.make_async_copy(
            olk[u], lk_hbm.at[pl.ds(0, _CHUNK)], slk[u]).wait()


@functools.cache
def _sc_call():
    buf = pltpu.VMEM((_CHUNK,), jnp.float32)
    sem = pltpu.SemaphoreType.DMA
    return pl.kernel(
        _sc_body,
        out_type=jax.ShapeDtypeStruct((_B * _C * _N,), jnp.float32),
        mesh=plsc.VectorSubcoreMesh(core_axis_name="c", subcore_axis_name="s",
                                    num_cores=_NC, num_subcores=_NS),
        compiler_params=pltpu.CompilerParams(use_tc_tiling_on_sc=False,
                                             needs_layout_passes=False),
        scratch_types=[
            pltpu.VMEM((_C * _LUT,), jnp.float32),
            pltpu.VMEM((_C * 64,), jnp.float32),
            buf, buf, buf, buf,
            sem, sem, sem, sem,
        ],
    )


@jax.jit
def kernel(x, H0, H1, H2, H3, H4, b0, b1, b2, b3, b4, a0, a1, a2, a3):
    B, C, H, W = x.shape
    N = H * W
    xr = x.reshape(B * C * N)
    wraw = jnp.concatenate(
        [H0.reshape(C, 3), H1.reshape(C, 9), H2.reshape(C, 9),
         H3.reshape(C, 9), H4.reshape(C, 3),
         b0.reshape(C, 3), b1.reshape(C, 3), b2.reshape(C, 3),
         b3.reshape(C, 3), b4.reshape(C, 1),
         a0.reshape(C, 3), a1.reshape(C, 3), a2.reshape(C, 3),
         a3.reshape(C, 3), jnp.zeros((C, 6), jnp.float32)], axis=1)
    lut, wproc = _build_tables(wraw)
    lk = _sc_call()(xr, lut.reshape(-1), wproc.reshape(-1))
    xh = _round_tc(x)
    return xh, lk.reshape(B, C, H, W)


# speedup vs baseline: 2.2265x; 2.2265x over previous
import functools

import jax
import jax.numpy as jnp
from jax import lax
from jax.experimental import pallas as pl
from jax.experimental.pallas import tpu as pltpu
from jax.experimental.pallas import tpu_sc as plsc

_C = 192
_N = 1024
_B = 32
_NC = 2
_NS = 16
_NW = _NC * _NS
_ROWS_PC = 8
_CHUNK = _ROWS_PC * _N
_NCH = _C // _ROWS_PC
_LUT = 256
_MAGIC = 12582912.0

_SP0, _SP1, _SP2, _SP3, _SP4 = 0, 3, 12, 21, 30
_B0, _B1, _B2, _B3, _B4 = 33, 36, 39, 42, 45
_T0, _T1, _T2, _T3 = 46, 49, 52, 55


def _builder_body(wraw_ref, lut_ref, wproc_ref):
    wr = wraw_ref[...]
    col = lax.broadcasted_iota(jnp.int32, wr.shape, 1)
    wp = jnp.where(col < _B0, jax.nn.softplus(wr),
                   jnp.where(col < _T0, wr, jnp.tanh(wr)))
    wproc_ref[...] = wp

    def sl(j):
        return wp[:, j:j + 1]

    def logits(u):
        l = [sl(_SP0 + j) * u + sl(_B0 + j) for j in range(3)]
        l = [l[j] + sl(_T0 + j) * jnp.tanh(l[j]) for j in range(3)]
        for (wc, bc, tc) in ((_SP1, _B1, _T1), (_SP2, _B2, _T2),
                             (_SP3, _B3, _T3)):
            nl = [sl(wc + 3 * j) * l[0] + sl(wc + 3 * j + 1) * l[1]
                  + sl(wc + 3 * j + 2) * l[2] + sl(bc + j) for j in range(3)]
            l = [nl[j] + sl(tc + j) * jnp.tanh(nl[j]) for j in range(3)]
        return sl(_SP4) * l[0] + sl(_SP4 + 1) * l[1] + sl(_SP4 + 2) * l[2] \
            + sl(_B4)

    vh = lax.broadcasted_iota(
        jnp.int32, (wr.shape[0], _LUT), 1).astype(jnp.float32) - 128.0
    lower = logits(vh - 0.5)
    upper = logits(vh + 0.5)
    s = -jnp.sign(lower + upper)
    lk = jnp.abs(jax.nn.sigmoid(s * upper) - jax.nn.sigmoid(s * lower))
    lut_ref[...] = jnp.maximum(lk, 1e-9)


def _build_tables(wraw):
    return pl.pallas_call(
        _builder_body,
        out_shape=[jax.ShapeDtypeStruct((_C, _LUT), jnp.float32),
                   jax.ShapeDtypeStruct((_C, 64), jnp.float32)],
    )(wraw)


def _tanh_e(t):
    return 1.0 - 2.0 / (jnp.exp(2.0 * t) + 1.0)


def _sigmoid_e(t):
    return 1.0 / (1.0 + jnp.exp(-t))


def _exact_round(v):
    m1 = (v + _MAGIC) - _MAGIC
    ki = (2.0 * v).astype(jnp.int32)
    odd = jnp.bitwise_and(ki, 1)
    mid = jnp.where(odd == 1, ((ki + 2) >> 2) << 1, ki >> 1)
    mid = mid.astype(jnp.float32)
    av = jnp.abs(v)
    return jnp.where(av <= 4194304.0, m1,
                     jnp.where(av < 8388608.0, mid, v))


def _mlp(wv, u):
    def g(c):
        return wv[c // 16][c % 16]

    l = [g(_SP0 + j) * u + g(_B0 + j) for j in range(3)]
    l = [l[j] + g(_T0 + j) * _tanh_e(l[j]) for j in range(3)]
    for (wc, bc, tc) in ((_SP1, _B1, _T1), (_SP2, _B2, _T2), (_SP3, _B3, _T3)):
        nl = [g(wc + 3 * j) * l[0] + g(wc + 3 * j + 1) * l[1]
              + g(wc + 3 * j + 2) * l[2] + g(bc + j)
              for j in range(3)]
        l = [nl[j] + g(tc + j) * _tanh_e(nl[j]) for j in range(3)]
    return (g(_SP4) * l[0] + g(_SP4 + 1) * l[1]
            + g(_SP4 + 2) * l[2] + g(_B4))


def _sc_body(x_hbm, lut_hbm, w_hbm, xh_hbm, lk_hbm,
             lut_v, w_v, xin0, xin1, oxh0, oxh1, olk0, olk1,
             si0, si1, sh0, sh1, sl0, sl1):
    cid = lax.axis_index("c")
    sid = lax.axis_index("s")
    wid = sid * _NC + cid
    base = wid * (_C * _N)
    pltpu.sync_copy(lut_hbm, lut_v)
    pltpu.sync_copy(w_hbm, w_v)

    xin = (xin0, xin1)
    oxh = (oxh0, oxh1)
    olk = (olk0, olk1)
    sin = (si0, si1)
    sxh = (sh0, sh1)
    slk = (sl0, sl1)

    def start_in(t, b):
        pltpu.async_copy(x_hbm.at[pl.ds(base + t * _CHUNK, _CHUNK)],
                         xin[b], sin[b])

    def compute(t, b):
        xv, hv, lv = xin[b], oxh[b], olk[b]
        r0 = t * _ROWS_PC

        @plsc.parallel_loop(0, _CHUNK // 16, unroll=8,
                            carry=jnp.zeros((16,), jnp.float32))
        def macc(i, acc):
            off = i * 16
            ch = r0 + (i >> 6)
            chb = jnp.broadcast_to(ch << 8, (16,)) + 128
            v = xv[pl.ds(off, 16)]
            acc = jnp.maximum(acc, jnp.abs(v))
            vh = (v + _MAGIC) - _MAGIC
            hv[pl.ds(off, 16)] = vh
            ci = jnp.clip(vh, -128.0, 127.0).astype(jnp.int32) + chb
            lv[pl.ds(off, 16)] = plsc.load_gather(lut_v, [ci])
            return acc

        m = jnp.max(macc)

        @pl.when(m > 127.0)
        def _slow():
            def row_slow(r, carry2):
                ch = r0 + r
                wv = tuple(w_v[pl.ds(ch * 64 + k * 16, 16)]
                           for k in range(4))
                rb = r * _N

                def slow(i, carry3):
                    off = rb + i * 16
                    v = xv[pl.ds(off, 16)]
                    vh = _exact_round(v)
                    hv[pl.ds(off, 16)] = vh
                    lo = _mlp(wv, vh - 0.5)
                    up = _mlp(wv, vh + 0.5)
                    s = -jnp.sign(lo + up)
                    lk = jnp.abs(_sigmoid_e(s * up) - _sigmoid_e(s * lo))
                    lv[pl.ds(off, 16)] = jnp.maximum(lk, 1e-9)
                    return carry3

                return lax.fori_loop(0, _N // 16, slow, carry2)

            lax.fori_loop(0, _ROWS_PC, row_slow, 0)

    start_in(0, 0)

    def gbody(g, carry):
        for u in (0, 1):
            t = 2 * g + u

            @pl.when(t + 1 < _NCH)
            def _pref():
                start_in(t + 1, 1 - u)

            pltpu.make_async_copy(
                x_hbm.at[pl.ds(0, _CHUNK)], xin[u], sin[u]).wait()

            @pl.when(g >= 1)
            def _wait_out():
                pltpu.make_async_copy(
                    oxh[u], xh_hbm.at[pl.ds(0, _CHUNK)], sxh[u]).wait()
                pltpu.make_async_copy(
                    olk[u], lk_hbm.at[pl.ds(0, _CHUNK)], slk[u]).wait()

            compute(t, u)
            pltpu.async_copy(
                oxh[u], xh_hbm.at[pl.ds(base + t * _CHUNK, _CHUNK)], sxh[u])
            pltpu.async_copy(
                olk[u], lk_hbm.at[pl.ds(base + t * _CHUNK, _CHUNK)], slk[u])
        return carry

    lax.fori_loop(0, _NCH // 2, gbody, 0)
    for u in (0, 1):
        pltpu.make_async_copy(
            oxh[u], xh_hbm.at[pl.ds(0, _CHUNK)], sxh[u]).wait()
        pltpu.make_async_copy(
            olk[u], lk_hbm.at[pl.ds(0, _CHUNK)], slk[u]).wait()


@functools.cache
def _sc_call():
    buf = pltpu.VMEM((_CHUNK,), jnp.float32)
    sem = pltpu.SemaphoreType.DMA
    return pl.kernel(
        _sc_body,
        out_type=[jax.ShapeDtypeStruct((_B * _C * _N,), jnp.float32),
                  jax.ShapeDtypeStruct((_B * _C * _N,), jnp.float32)],
        mesh=plsc.VectorSubcoreMesh(core_axis_name="c", subcore_axis_name="s",
                                    num_cores=_NC, num_subcores=_NS),
        compiler_params=pltpu.CompilerParams(use_tc_tiling_on_sc=False,
                                             needs_layout_passes=False),
        scratch_types=[
            pltpu.VMEM((_C * _LUT,), jnp.float32),
            pltpu.VMEM((_C * 64,), jnp.float32),
            buf, buf, buf, buf, buf, buf,
            sem, sem, sem, sem, sem, sem,
        ],
    )


@jax.jit
def kernel(x, H0, H1, H2, H3, H4, b0, b1, b2, b3, b4, a0, a1, a2, a3):
    B, C, H, W = x.shape
    N = H * W
    xr = x.reshape(B * C * N)
    wraw = jnp.concatenate(
        [H0.reshape(C, 3), H1.reshape(C, 9), H2.reshape(C, 9),
         H3.reshape(C, 9), H4.reshape(C, 3),
         b0.reshape(C, 3), b1.reshape(C, 3), b2.reshape(C, 3),
         b3.reshape(C, 3), b4.reshape(C, 1),
         a0.reshape(C, 3), a1.reshape(C, 3), a2.reshape(C, 3),
         a3.reshape(C, 3), jnp.zeros((C, 6), jnp.float32)], axis=1)
    lut, wproc = _build_tables(wraw)
    xh, lk = _sc_call()(xr, lut.reshape(-1), wproc.reshape(-1))
    return xh.reshape(B, C, H, W), lk.reshape(B, C, H, W)
